# baseline (speedup 1.0000x reference)
import functools

import jax
import jax.numpy as jnp
from jax import lax
from jax.experimental import pallas as pl
from jax.experimental.pallas import tpu as pltpu
from jax.experimental.pallas import tpu_sc as plsc

DIM = 128
NC = 2
NS = 16
NW = NC * NS
B_TOTAL = 4 * 8192
B_PER_W = B_TOTAL // NW
CHUNK = 128
N_CHUNKS = B_PER_W // CHUNK

SUP = 2
N_SUP = N_CHUNKS // SUP
NBUF = 3

_mesh = plsc.VectorSubcoreMesh(core_axis_name="c", subcore_axis_name="s")


@functools.partial(
    pl.kernel,
    mesh=_mesh,
    out_type=jax.ShapeDtypeStruct((2, B_TOTAL, DIM), jnp.float32),
    scratch_types=[
        pltpu.VMEM((N_CHUNKS, CHUNK), jnp.int32),
        *([pltpu.VMEM((SUP * CHUNK, DIM), jnp.float32)] * NBUF),
        *([pltpu.SemaphoreType.DMA] * (2 * NBUF)),
    ],
)
def _rope_gather(cos_hbm, sin_hbm, idx_hbm, out_hbm, idx_v, *bufs_and_sems):
    bufs = bufs_and_sems[:NBUF]
    gsems = bufs_and_sems[NBUF:2 * NBUF]
    ssems = bufs_and_sems[2 * NBUF:]
    wid = lax.axis_index("s") * NC + lax.axis_index("c")
    base = wid * B_PER_W
    pltpu.sync_copy(idx_hbm.at[wid // 8, pl.ds((wid % 8) * 8, N_CHUNKS)], idx_v)
    tables = (cos_hbm, sin_hbm)
    items = [(t, p) for p in range(N_SUP) for t in range(2)]
    n = len(items)

    def issue_gathers(item, b):
        t, p = item
        return [
            pltpu.async_copy(tables[t].at[idx_v.at[SUP * p + u]],
                             bufs[b].at[pl.ds(u * CHUNK, CHUNK)], gsems[b])
            for u in range(SUP)
        ]

    gd = [None] * NBUF
    sd = [None] * NBUF
    for j in range(min(NBUF - 1, n)):
        gd[j] = issue_gathers(items[j], j)
    for i in range(n):
        b = i % NBUF
        j = i + NBUF - 1
        if j < n:
            jb = j % NBUF
            if sd[jb] is not None:
                sd[jb].wait()
            gd[jb] = issue_gathers(items[j], jb)
        for d in gd[b]:
            d.wait()
        t, p = items[i]
        sd[b] = pltpu.async_copy(
            bufs[b], out_hbm.at[t, pl.ds(base + p * SUP * CHUNK, SUP * CHUNK)], ssems[b])
    for b in range(NBUF):
        if sd[b] is not None:
            sd[b].wait()


def kernel(cos_cached, sin_cached, position_ids):
    idx = position_ids.reshape(4, 64, CHUNK)
    out = _rope_gather(cos_cached, sin_cached, idx)
    return out.reshape(2, 4, 8192, DIM)


# speedup vs baseline: 1.0223x; 1.0223x over previous
import functools

import jax
import jax.numpy as jnp
from jax import lax
from jax.experimental import pallas as pl
from jax.experimental.pallas import tpu as pltpu
from jax.experimental.pallas import tpu_sc as plsc

DIM = 128
NC = 2
NS = 16
NW = NC * NS
B_TOTAL = 4 * 8192
B_PER_W = B_TOTAL // NW
CHUNK = 128
N_CHUNKS = B_PER_W // CHUNK

SUP = 1
N_SUP = N_CHUNKS // SUP
NBUF = 6

_mesh = plsc.VectorSubcoreMesh(core_axis_name="c", subcore_axis_name="s")


@functools.partial(
    pl.kernel,
    mesh=_mesh,
    out_type=jax.ShapeDtypeStruct((2, B_TOTAL, DIM), jnp.float32),
    scratch_types=[
        pltpu.VMEM((N_CHUNKS, CHUNK), jnp.int32),
        *([pltpu.VMEM((SUP * CHUNK, DIM), jnp.float32)] * NBUF),
        *([pltpu.SemaphoreType.DMA] * (2 * NBUF)),
    ],
)
def _rope_gather(cos_hbm, sin_hbm, idx_hbm, out_hbm, idx_v, *bufs_and_sems):
    bufs = bufs_and_sems[:NBUF]
    gsems = bufs_and_sems[NBUF:2 * NBUF]
    ssems = bufs_and_sems[2 * NBUF:]
    wid = lax.axis_index("s") * NC + lax.axis_index("c")
    base = wid * B_PER_W
    pltpu.sync_copy(idx_hbm.at[wid // 8, pl.ds((wid % 8) * 8, N_CHUNKS)], idx_v)
    tables = (cos_hbm, sin_hbm)
    items = [(t, p) for p in range(N_SUP) for t in range(2)]
    n = len(items)

    def issue_gathers(item, b):
        t, p = item
        return [
            pltpu.async_copy(tables[t].at[idx_v.at[SUP * p + u]],
                             bufs[b].at[pl.ds(u * CHUNK, CHUNK)], gsems[b])
            for u in range(SUP)
        ]

    gd = [None] * NBUF
    sd = [None] * NBUF
    for j in range(min(NBUF - 1, n)):
        gd[j] = issue_gathers(items[j], j)
    for i in range(n):
        b = i % NBUF
        j = i + NBUF - 1
        if j < n:
            jb = j % NBUF
            if sd[jb] is not None:
                sd[jb].wait()
            gd[jb] = issue_gathers(items[j], jb)
        for d in gd[b]:
            d.wait()
        t, p = items[i]
        sd[b] = pltpu.async_copy(
            bufs[b], out_hbm.at[t, pl.ds(base + p * SUP * CHUNK, SUP * CHUNK)], ssems[b])
    for b in range(NBUF):
        if sd[b] is not None:
            sd[b].wait()


def kernel(cos_cached, sin_cached, position_ids):
    idx = position_ids.reshape(4, 64, CHUNK)
    out = _rope_gather(cos_cached, sin_cached, idx)
    return out.reshape(2, 4, 8192, DIM)
